# baseline (speedup 1.0000x reference)
import jax
import jax.numpy as jnp
from jax import lax
from jax.experimental import pallas as pl
from jax.experimental.pallas import tpu as pltpu
from jax.experimental.pallas import tpu_sc as plsc

B = 64
N = 65536
IN_DIM = 8192
OUT_DIM = 8192

NC = 2
NS = 16
NW = NC * NS
ROWS_PER_W = B // NW
CHUNK = 16384
N_CHUNKS = N // CHUNK
L = 16


def _body(x_hbm, ind_hbm, val_hbm, bias_hbm, out_hbm,
          x_v, y_v, oi_v0, oi_v1, ii_v0, ii_v1, val_v0, val_v1, sem0, sem1):
    wid = lax.axis_index("s") * NC + lax.axis_index("c")
    oi_bufs = [oi_v0, oi_v1]
    ii_bufs = [ii_v0, ii_v1]
    val_bufs = [val_v0, val_v1]
    sem_bufs = [sem0, sem1]

    def start_chunk(b, c, p):
        sl = pl.ds(c * CHUNK, CHUNK)
        ho = pltpu.async_copy(ind_hbm.at[b, 0, sl], oi_bufs[p], sem_bufs[p])
        hi = pltpu.async_copy(ind_hbm.at[b, 1, sl], ii_bufs[p], sem_bufs[p])
        hv = pltpu.async_copy(val_hbm.at[b, sl], val_bufs[p], sem_bufs[p])
        return ho, hi, hv

    for bb in range(ROWS_PER_W):
        b = wid * ROWS_PER_W + bb
        pltpu.sync_copy(x_hbm.at[b], x_v)
        pltpu.sync_copy(bias_hbm, y_v)
        pending = start_chunk(b, 0, 0)

        for c in range(N_CHUNKS):
            p = c % 2
            for h in pending:
                h.wait()
            if c + 1 < N_CHUNKS:
                pending = start_chunk(b, c + 1, 1 - p)
            oi_c = oi_bufs[p]
            ii_c = ii_bufs[p]
            val_c = val_bufs[p]

            @plsc.parallel_loop(0, CHUNK // L, unroll=16)
            def _grp(j):
                oi = oi_c[pl.ds(j * L, L)]
                ii = ii_c[pl.ds(j * L, L)]
                v = val_c[pl.ds(j * L, L)]
                of_i = oi.astype(jnp.int32)
                fi_i = ii.astype(jnp.int32)
                fr_o = oi - of_i.astype(jnp.float32)
                fr_i = ii - fi_i.astype(jnp.float32)
                wo_f = jnp.where(fr_o > 0.0, 1.0 - fr_o, 2.0)
                wi_f = jnp.where(fr_i > 0.0, 1.0 - fr_i, 2.0)
                g = wi_f * plsc.load_gather(x_v, [fi_i]) \
                    + fr_i * plsc.load_gather(x_v, [fi_i + 1])
                vg = v * g
                plsc.addupdate_scatter(y_v, [of_i], wo_f * vg)
                plsc.addupdate_scatter(y_v, [of_i + 1], fr_o * vg)

        pltpu.sync_copy(y_v, out_hbm.at[b])


@jax.jit
def kernel(x, real_indices, real_values, bias):
    mesh = plsc.VectorSubcoreMesh(core_axis_name="c", subcore_axis_name="s")
    run = pl.kernel(
        _body,
        out_type=jax.ShapeDtypeStruct((B, OUT_DIM), jnp.float32),
        mesh=mesh,
        scratch_types=[
            pltpu.VMEM((IN_DIM,), jnp.float32),
            pltpu.VMEM((OUT_DIM,), jnp.float32),
            pltpu.VMEM((CHUNK,), jnp.float32),
            pltpu.VMEM((CHUNK,), jnp.float32),
            pltpu.VMEM((CHUNK,), jnp.float32),
            pltpu.VMEM((CHUNK,), jnp.float32),
            pltpu.VMEM((CHUNK,), jnp.float32),
            pltpu.VMEM((CHUNK,), jnp.float32),
            pltpu.SemaphoreType.DMA,
            pltpu.SemaphoreType.DMA,
        ],
        compiler_params=pltpu.CompilerParams(needs_layout_passes=False),
    )
    return run(x, real_indices.transpose(0, 2, 1), real_values, bias)


# speedup vs baseline: 2.3344x; 2.3344x over previous
import jax
import jax.numpy as jnp
from jax import lax
from jax.experimental import pallas as pl
from jax.experimental.pallas import tpu as pltpu
from jax.experimental.pallas import tpu_sc as plsc

B = 64
N = 65536
IN_DIM = 8192
OUT_DIM = 8192

NC = 2
NS = 16
NW = NC * NS
ROWS_PER_W = B // NW
CHUNK = 8192
N_CHUNKS = N // CHUNK
L = 16


def _body(x_hbm, ind_hbm, val_hbm, bias_hbm, out_hbm,
          x_v0, x_v1, y_v0, y_v1,
          oi_v0, oi_v1, ii_v0, ii_v1, val_v0, val_v1,
          sem0, sem1, xsem0, xsem1, wsem0, wsem1):
    wid = lax.axis_index("s") * NC + lax.axis_index("c")
    xb = [x_v0, x_v1]
    yb = [y_v0, y_v1]
    oi_bufs = [oi_v0, oi_v1]
    ii_bufs = [ii_v0, ii_v1]
    val_bufs = [val_v0, val_v1]
    sem_bufs = [sem0, sem1]
    xsems = [xsem0, xsem1]
    wsems = [wsem0, wsem1]

    def start_chunk(b, c, p):
        sl = pl.ds(c * CHUNK, CHUNK)
        ho = pltpu.async_copy(ind_hbm.at[b, 0, sl], oi_bufs[p], sem_bufs[p])
        hi = pltpu.async_copy(ind_hbm.at[b, 1, sl], ii_bufs[p], sem_bufs[p])
        hv = pltpu.async_copy(val_hbm.at[b, sl], val_bufs[p], sem_bufs[p])
        return ho, hi, hv

    def start_row(r):
        b = wid * ROWS_PER_W + r
        hx = pltpu.async_copy(x_hbm.at[b], xb[r], xsems[r])
        hbias = pltpu.async_copy(bias_hbm, yb[r], xsems[r])
        return hx, hbias

    tasks = [(r, wid * ROWS_PER_W + r, c)
             for r in range(ROWS_PER_W) for c in range(N_CHUNKS)]

    row_pending = [start_row(r) for r in range(ROWS_PER_W)]
    pending = [start_chunk(tasks[0][1], tasks[0][2], 0),
               start_chunk(tasks[1][1], tasks[1][2], 1)]
    writebacks = []

    for t, (r, b, c) in enumerate(tasks):
        p = t % 2
        for h in pending[p]:
            h.wait()
        if t + 2 < len(tasks):
            nr, nb, nck = tasks[t + 2]
            pending[p] = start_chunk(nb, nck, p)
        if c == 0:
            for h in row_pending[r]:
                h.wait()
        x_v = xb[r]
        y_v = yb[r]
        oi_c = oi_bufs[p]
        ii_c = ii_bufs[p]
        val_c = val_bufs[p]

        @plsc.parallel_loop(0, CHUNK // L, unroll=8)
        def _grp(j):
            oi = oi_c[pl.ds(j * L, L)]
            ii = ii_c[pl.ds(j * L, L)]
            v = val_c[pl.ds(j * L, L)]
            of_i = oi.astype(jnp.int32)
            fi_i = ii.astype(jnp.int32)
            fr_o = oi - of_i.astype(jnp.float32)
            fr_i = ii - fi_i.astype(jnp.float32)
            wo_f = jnp.where(fr_o > 0.0, 1.0 - fr_o, 2.0)
            wi_f = jnp.where(fr_i > 0.0, 1.0 - fr_i, 2.0)
            g = wi_f * plsc.load_gather(x_v, [fi_i]) \
                + fr_i * plsc.load_gather(x_v, [fi_i + 1])
            vg = v * g
            plsc.addupdate_scatter(y_v, [of_i], wo_f * vg)
            plsc.addupdate_scatter(y_v, [of_i + 1], fr_o * vg)

        if c == N_CHUNKS - 1:
            writebacks.append(
                pltpu.async_copy(y_v, out_hbm.at[b], wsems[r]))

    for h in writebacks:
        h.wait()


@jax.jit
def kernel(x, real_indices, real_values, bias):
    mesh = plsc.VectorSubcoreMesh(core_axis_name="c", subcore_axis_name="s")
    run = pl.kernel(
        _body,
        out_type=jax.ShapeDtypeStruct((B, OUT_DIM), jnp.float32),
        mesh=mesh,
        scratch_types=[
            pltpu.VMEM((IN_DIM,), jnp.float32),
            pltpu.VMEM((IN_DIM,), jnp.float32),
            pltpu.VMEM((OUT_DIM,), jnp.float32),
            pltpu.VMEM((OUT_DIM,), jnp.float32),
            pltpu.VMEM((CHUNK,), jnp.float32),
            pltpu.VMEM((CHUNK,), jnp.float32),
            pltpu.VMEM((CHUNK,), jnp.float32),
            pltpu.VMEM((CHUNK,), jnp.float32),
            pltpu.VMEM((CHUNK,), jnp.float32),
            pltpu.VMEM((CHUNK,), jnp.float32),
            pltpu.SemaphoreType.DMA,
            pltpu.SemaphoreType.DMA,
            pltpu.SemaphoreType.DMA,
            pltpu.SemaphoreType.DMA,
            pltpu.SemaphoreType.DMA,
            pltpu.SemaphoreType.DMA,
        ],
        compiler_params=pltpu.CompilerParams(needs_layout_passes=False),
    )
    return run(x, real_indices.transpose(0, 2, 1), real_values, bias)


# speedup vs baseline: 2.3359x; 1.0006x over previous
import jax
import jax.numpy as jnp
from jax import lax
from jax.experimental import pallas as pl
from jax.experimental.pallas import tpu as pltpu
from jax.experimental.pallas import tpu_sc as plsc

B = 64
N = 65536
IN_DIM = 8192
OUT_DIM = 8192

NC = 2
NS = 16
NW = NC * NS
ROWS_PER_W = B // NW
CHUNK = 8192
N_CHUNKS = N // CHUNK
L = 16


def _body(x_hbm, ind_hbm, val_hbm, bias_hbm, out_hbm,
          x_v0, x_v1, y_v0, y_v1,
          oi_v0, oi_v1, oi_v2, ii_v0, ii_v1, ii_v2,
          val_v0, val_v1, val_v2,
          sem0, sem1, sem2, xsem0, xsem1, wsem0, wsem1):
    wid = lax.axis_index("s") * NC + lax.axis_index("c")
    xb = [x_v0, x_v1]
    yb = [y_v0, y_v1]
    oi_bufs = [oi_v0, oi_v1, oi_v2]
    ii_bufs = [ii_v0, ii_v1, ii_v2]
    val_bufs = [val_v0, val_v1, val_v2]
    sem_bufs = [sem0, sem1, sem2]
    xsems = [xsem0, xsem1]
    wsems = [wsem0, wsem1]

    def start_chunk(b, c, p):
        sl = pl.ds(c * CHUNK, CHUNK)
        ho = pltpu.async_copy(ind_hbm.at[b, 0, sl], oi_bufs[p], sem_bufs[p])
        hi = pltpu.async_copy(ind_hbm.at[b, 1, sl], ii_bufs[p], sem_bufs[p])
        hv = pltpu.async_copy(val_hbm.at[b, sl], val_bufs[p], sem_bufs[p])
        return ho, hi, hv

    def start_row(r):
        b = wid * ROWS_PER_W + r
        hx = pltpu.async_copy(x_hbm.at[b], xb[r], xsems[r])
        hbias = pltpu.async_copy(bias_hbm, yb[r], xsems[r])
        return hx, hbias

    tasks = [(r, wid * ROWS_PER_W + r, c)
             for r in range(ROWS_PER_W) for c in range(N_CHUNKS)]

    row_pending = [start_row(r) for r in range(ROWS_PER_W)]
    pending = [start_chunk(tasks[0][1], tasks[0][2], 0),
               start_chunk(tasks[1][1], tasks[1][2], 1),
               None]
    writebacks = []

    for t, (r, b, c) in enumerate(tasks):
        p = t % 3
        for h in pending[p]:
            h.wait()
        if t + 2 < len(tasks):
            nr, nb, nck = tasks[t + 2]
            pending[(t + 2) % 3] = start_chunk(nb, nck, (t + 2) % 3)
        if c == 0:
            for h in row_pending[r]:
                h.wait()
        x_v = xb[r]
        y_v = yb[r]
        oi_c = oi_bufs[p]
        ii_c = ii_bufs[p]
        val_c = val_bufs[p]

        @plsc.parallel_loop(0, CHUNK // L, unroll=8)
        def _grp(j):
            oi = oi_c[pl.ds(j * L, L)]
            ii = ii_c[pl.ds(j * L, L)]
            v = val_c[pl.ds(j * L, L)]
            of_i = oi.astype(jnp.int32)
            fi_i = ii.astype(jnp.int32)
            fr_o = oi - of_i.astype(jnp.float32)
            fr_i = ii - fi_i.astype(jnp.float32)
            wo_f = jnp.where(fr_o > 0.0, 1.0 - fr_o, 2.0)
            wi_f = jnp.where(fr_i > 0.0, 1.0 - fr_i, 2.0)
            g = wi_f * plsc.load_gather(x_v, [fi_i]) \
                + fr_i * plsc.load_gather(x_v, [fi_i + 1])
            vg = v * g
            plsc.addupdate_scatter(y_v, [of_i], wo_f * vg)
            plsc.addupdate_scatter(y_v, [of_i + 1], fr_o * vg)

        if c == N_CHUNKS - 1:
            writebacks.append(
                pltpu.async_copy(y_v, out_hbm.at[b], wsems[r]))

    for h in writebacks:
        h.wait()


@jax.jit
def kernel(x, real_indices, real_values, bias):
    mesh = plsc.VectorSubcoreMesh(core_axis_name="c", subcore_axis_name="s")
    run = pl.kernel(
        _body,
        out_type=jax.ShapeDtypeStruct((B, OUT_DIM), jnp.float32),
        mesh=mesh,
        scratch_types=[
            pltpu.VMEM((IN_DIM,), jnp.float32),
            pltpu.VMEM((IN_DIM,), jnp.float32),
            pltpu.VMEM((OUT_DIM,), jnp.float32),
            pltpu.VMEM((OUT_DIM,), jnp.float32),
            pltpu.VMEM((CHUNK,), jnp.float32),
            pltpu.VMEM((CHUNK,), jnp.float32),
            pltpu.VMEM((CHUNK,), jnp.float32),
            pltpu.VMEM((CHUNK,), jnp.float32),
            pltpu.VMEM((CHUNK,), jnp.float32),
            pltpu.VMEM((CHUNK,), jnp.float32),
            pltpu.VMEM((CHUNK,), jnp.float32),
            pltpu.VMEM((CHUNK,), jnp.float32),
            pltpu.VMEM((CHUNK,), jnp.float32),
            pltpu.SemaphoreType.DMA,
            pltpu.SemaphoreType.DMA,
            pltpu.SemaphoreType.DMA,
            pltpu.SemaphoreType.DMA,
            pltpu.SemaphoreType.DMA,
            pltpu.SemaphoreType.DMA,
            pltpu.SemaphoreType.DMA,
        ],
        compiler_params=pltpu.CompilerParams(needs_layout_passes=False),
    )
    return run(x, real_indices.transpose(0, 2, 1), real_values, bias)


# speedup vs baseline: 2.3526x; 1.0071x over previous
import jax
import jax.numpy as jnp
from jax import lax
from jax.experimental import pallas as pl
from jax.experimental.pallas import tpu as pltpu
from jax.experimental.pallas import tpu_sc as plsc

B = 64
N = 65536
IN_DIM = 8192
OUT_DIM = 8192

NC = 2
NS = 16
NW = NC * NS
ROWS_PER_W = B // NW
CHUNK = 8192
N_CHUNKS = N // CHUNK
L = 16


def _body(x_hbm, ind_hbm, val_hbm, bias_hbm, out_hbm,
          x_v0, x_v1, y_v0, y_v1,
          oi_v0, oi_v1, oi_v2, ii_v0, ii_v1, ii_v2,
          val_v0, val_v1, val_v2,
          sem0, sem1, sem2, xsem0, xsem1, wsem0, wsem1):
    wid = lax.axis_index("s") * NC + lax.axis_index("c")
    xb = [x_v0, x_v1]
    yb = [y_v0, y_v1]
    oi_bufs = [oi_v0, oi_v1, oi_v2]
    ii_bufs = [ii_v0, ii_v1, ii_v2]
    val_bufs = [val_v0, val_v1, val_v2]
    sem_bufs = [sem0, sem1, sem2]
    xsems = [xsem0, xsem1]
    wsems = [wsem0, wsem1]

    def start_chunk(b, c, p):
        sl = pl.ds(c * CHUNK, CHUNK)
        ho = pltpu.async_copy(ind_hbm.at[b, 0, sl], oi_bufs[p], sem_bufs[p])
        hi = pltpu.async_copy(ind_hbm.at[b, 1, sl], ii_bufs[p], sem_bufs[p])
        hv = pltpu.async_copy(val_hbm.at[b, sl], val_bufs[p], sem_bufs[p])
        return ho, hi, hv

    def start_row(r):
        b = wid * ROWS_PER_W + r
        hx = pltpu.async_copy(x_hbm.at[b], xb[r], xsems[r])
        hbias = pltpu.async_copy(bias_hbm, yb[r], xsems[r])
        return hx, hbias

    tasks = [(r, wid * ROWS_PER_W + r, c)
             for r in range(ROWS_PER_W) for c in range(N_CHUNKS)]

    row_pending = [start_row(r) for r in range(ROWS_PER_W)]
    pending = [start_chunk(tasks[0][1], tasks[0][2], 0),
               start_chunk(tasks[1][1], tasks[1][2], 1),
               None]
    writebacks = []

    for t, (r, b, c) in enumerate(tasks):
        p = t % 3
        for h in pending[p]:
            h.wait()
        if t + 2 < len(tasks):
            nr, nb, nck = tasks[t + 2]
            pending[(t + 2) % 3] = start_chunk(nb, nck, (t + 2) % 3)
        if c == 0:
            for h in row_pending[r]:
                h.wait()
        x_v = xb[r]
        y_v = yb[r]
        oi_c = oi_bufs[p]
        ii_c = ii_bufs[p]
        val_c = val_bufs[p]

        @plsc.parallel_loop(0, CHUNK // L, unroll=4)
        def _grp(j):
            oi = oi_c[pl.ds(j * L, L)]
            ii = ii_c[pl.ds(j * L, L)]
            v = val_c[pl.ds(j * L, L)]
            of_i = oi.astype(jnp.int32)
            fi_i = ii.astype(jnp.int32)
            fr_o = oi - of_i.astype(jnp.float32)
            fr_i = ii - fi_i.astype(jnp.float32)
            wo_f = jnp.where(fr_o > 0.0, 1.0 - fr_o, 2.0)
            wi_f = jnp.where(fr_i > 0.0, 1.0 - fr_i, 2.0)
            g = wi_f * plsc.load_gather(x_v, [fi_i]) \
                + fr_i * plsc.load_gather(x_v, [fi_i + 1])
            vg = v * g
            plsc.addupdate_scatter(y_v, [of_i], wo_f * vg)
            plsc.addupdate_scatter(y_v, [of_i + 1], fr_o * vg)

        if c == N_CHUNKS - 1:
            writebacks.append(
                pltpu.async_copy(y_v, out_hbm.at[b], wsems[r]))

    for h in writebacks:
        h.wait()


@jax.jit
def kernel(x, real_indices, real_values, bias):
    mesh = plsc.VectorSubcoreMesh(core_axis_name="c", subcore_axis_name="s")
    run = pl.kernel(
        _body,
        out_type=jax.ShapeDtypeStruct((B, OUT_DIM), jnp.float32),
        mesh=mesh,
        scratch_types=[
            pltpu.VMEM((IN_DIM,), jnp.float32),
            pltpu.VMEM((IN_DIM,), jnp.float32),
            pltpu.VMEM((OUT_DIM,), jnp.float32),
            pltpu.VMEM((OUT_DIM,), jnp.float32),
            pltpu.VMEM((CHUNK,), jnp.float32),
            pltpu.VMEM((CHUNK,), jnp.float32),
            pltpu.VMEM((CHUNK,), jnp.float32),
            pltpu.VMEM((CHUNK,), jnp.float32),
            pltpu.VMEM((CHUNK,), jnp.float32),
            pltpu.VMEM((CHUNK,), jnp.float32),
            pltpu.VMEM((CHUNK,), jnp.float32),
            pltpu.VMEM((CHUNK,), jnp.float32),
            pltpu.VMEM((CHUNK,), jnp.float32),
            pltpu.SemaphoreType.DMA,
            pltpu.SemaphoreType.DMA,
            pltpu.SemaphoreType.DMA,
            pltpu.SemaphoreType.DMA,
            pltpu.SemaphoreType.DMA,
            pltpu.SemaphoreType.DMA,
            pltpu.SemaphoreType.DMA,
        ],
        compiler_params=pltpu.CompilerParams(needs_layout_passes=False),
    )
    return run(x, real_indices.transpose(0, 2, 1), real_values, bias)


# speedup vs baseline: 2.3662x; 1.0058x over previous
import jax
import jax.numpy as jnp
from jax import lax
from jax.experimental import pallas as pl
from jax.experimental.pallas import tpu as pltpu
from jax.experimental.pallas import tpu_sc as plsc

B = 64
N = 65536
IN_DIM = 8192
OUT_DIM = 8192

NC = 2
NS = 16
NW = NC * NS
ROWS_PER_W = B // NW
CHUNK = 8192
N_CHUNKS = N // CHUNK
L = 16


def _body(x_hbm, ind_hbm, val_hbm, bias_hbm, out_hbm,
          x_v0, x_v1, y_v0, y_v1,
          oi_v0, oi_v1, oi_v2, ii_v0, ii_v1, ii_v2,
          val_v0, val_v1, val_v2,
          sem0, sem1, sem2, xsem0, xsem1, wsem0, wsem1):
    wid = lax.axis_index("s") * NC + lax.axis_index("c")
    xb = [x_v0, x_v1]
    yb = [y_v0, y_v1]
    oi_bufs = [oi_v0, oi_v1, oi_v2]
    ii_bufs = [ii_v0, ii_v1, ii_v2]
    val_bufs = [val_v0, val_v1, val_v2]
    sem_bufs = [sem0, sem1, sem2]
    xsems = [xsem0, xsem1]
    wsems = [wsem0, wsem1]

    def start_chunk(b, c, p):
        sl = pl.ds(c * CHUNK, CHUNK)
        ho = pltpu.async_copy(ind_hbm.at[b, 0, sl], oi_bufs[p], sem_bufs[p])
        hi = pltpu.async_copy(ind_hbm.at[b, 1, sl], ii_bufs[p], sem_bufs[p])
        hv = pltpu.async_copy(val_hbm.at[b, sl], val_bufs[p], sem_bufs[p])
        return ho, hi, hv

    def start_row(r):
        b = wid * ROWS_PER_W + r
        hx = pltpu.async_copy(x_hbm.at[b], xb[r], xsems[r])
        hbias = pltpu.async_copy(bias_hbm, yb[r], xsems[r])
        return hx, hbias

    tasks = [(r, wid * ROWS_PER_W + r, c)
             for r in range(ROWS_PER_W) for c in range(N_CHUNKS)]

    row_pending = [start_row(r) for r in range(ROWS_PER_W)]
    pending = [start_chunk(tasks[0][1], tasks[0][2], 0),
               start_chunk(tasks[1][1], tasks[1][2], 1),
               None]
    writebacks = []

    for t, (r, b, c) in enumerate(tasks):
        p = t % 3
        for h in pending[p]:
            h.wait()
        if t + 2 < len(tasks):
            nr, nb, nck = tasks[t + 2]
            pending[(t + 2) % 3] = start_chunk(nb, nck, (t + 2) % 3)
        if c == 0:
            for h in row_pending[r]:
                h.wait()
        x_v = xb[r]
        y_v = yb[r]
        oi_c = oi_bufs[p]
        ii_c = ii_bufs[p]
        val_c = val_bufs[p]

        @plsc.parallel_loop(0, CHUNK // L, unroll=6)
        def _grp(j):
            oi = oi_c[pl.ds(j * L, L)]
            ii = ii_c[pl.ds(j * L, L)]
            v = val_c[pl.ds(j * L, L)]
            of_i = oi.astype(jnp.int32)
            fi_i = ii.astype(jnp.int32)
            fr_o = oi - of_i.astype(jnp.float32)
            fr_i = ii - fi_i.astype(jnp.float32)
            wo_f = jnp.where(fr_o > 0.0, 1.0 - fr_o, 2.0)
            wi_f = jnp.where(fr_i > 0.0, 1.0 - fr_i, 2.0)
            g = wi_f * plsc.load_gather(x_v, [fi_i]) \
                + fr_i * plsc.load_gather(x_v, [fi_i + 1])
            vg = v * g
            plsc.addupdate_scatter(y_v, [of_i], wo_f * vg)
            plsc.addupdate_scatter(y_v, [of_i + 1], fr_o * vg)

        if c == N_CHUNKS - 1:
            writebacks.append(
                pltpu.async_copy(y_v, out_hbm.at[b], wsems[r]))

    for h in writebacks:
        h.wait()


@jax.jit
def kernel(x, real_indices, real_values, bias):
    mesh = plsc.VectorSubcoreMesh(core_axis_name="c", subcore_axis_name="s")
    run = pl.kernel(
        _body,
        out_type=jax.ShapeDtypeStruct((B, OUT_DIM), jnp.float32),
        mesh=mesh,
        scratch_types=[
            pltpu.VMEM((IN_DIM,), jnp.float32),
            pltpu.VMEM((IN_DIM,), jnp.float32),
            pltpu.VMEM((OUT_DIM,), jnp.float32),
            pltpu.VMEM((OUT_DIM,), jnp.float32),
            pltpu.VMEM((CHUNK,), jnp.float32),
            pltpu.VMEM((CHUNK,), jnp.float32),
            pltpu.VMEM((CHUNK,), jnp.float32),
            pltpu.VMEM((CHUNK,), jnp.float32),
            pltpu.VMEM((CHUNK,), jnp.float32),
            pltpu.VMEM((CHUNK,), jnp.float32),
            pltpu.VMEM((CHUNK,), jnp.float32),
            pltpu.VMEM((CHUNK,), jnp.float32),
            pltpu.VMEM((CHUNK,), jnp.float32),
            pltpu.SemaphoreType.DMA,
            pltpu.SemaphoreType.DMA,
            pltpu.SemaphoreType.DMA,
            pltpu.SemaphoreType.DMA,
            pltpu.SemaphoreType.DMA,
            pltpu.SemaphoreType.DMA,
            pltpu.SemaphoreType.DMA,
        ],
        compiler_params=pltpu.CompilerParams(needs_layout_passes=False),
    )
    return run(x, real_indices.transpose(0, 2, 1), real_values, bias)
